# baseline (speedup 1.0000x reference)
import functools

import jax
import jax.numpy as jnp
from jax import lax
from jax.experimental import pallas as pl
from jax.experimental.pallas import tpu as pltpu
from jax.experimental.pallas import tpu_sc as plsc

V = 65
B, T = 64, 2048
N = B * T

NC, NS, L = 2, 16, 16
NW = NC * NS
TPW = N // NW
CH = 256
NCH = TPW // CH
CPB = T // CH


def _nll_table_body(table_ref, nll_ref):
    x = table_ref[...]
    m = jnp.max(x, axis=-1, keepdims=True)
    lse = m + jnp.log(jnp.sum(jnp.exp(x - m), axis=-1, keepdims=True))
    nll_ref[...] = lse - x


_nll_table = pl.pallas_call(
    _nll_table_body,
    out_shape=jax.ShapeDtypeStruct((V, V), jnp.float32),
)


_sc_mesh = plsc.VectorSubcoreMesh(
    core_axis_name="c", subcore_axis_name="s", num_cores=NC, num_subcores=NS
)


@functools.partial(
    pl.kernel,
    out_type=(
        jax.ShapeDtypeStruct((B, T, V), jnp.float32),
        jax.ShapeDtypeStruct((NW, L), jnp.float32),
    ),
    mesh=_sc_mesh,
    compiler_params=pltpu.CompilerParams(
        needs_layout_passes=False, use_tc_tiling_on_sc=True
    ),
    scratch_types=[
        pltpu.VMEM((TPW,), jnp.int32),
        pltpu.VMEM((TPW,), jnp.int32),
        pltpu.VMEM((CH, V), jnp.float32),
        pltpu.VMEM((CH, V), jnp.float32),
        pltpu.VMEM((V * V,), jnp.float32),
        pltpu.VMEM((V * V,), jnp.float32),
        pltpu.VMEM((L,), jnp.float32),
        pltpu.SemaphoreType.DMA,
        pltpu.SemaphoreType.DMA,
    ],
)
def _sc_body(inp_hbm, tgt_hbm, tab_hbm, nll_hbm, out_hbm, part_hbm,
             idx_v, tgt_v, rows_a, rows_b, tab_v, nll_v, part_v,
             sem_a, sem_b):
    wid = lax.axis_index("s") * NC + lax.axis_index("c")
    base = wid * TPW

    pltpu.sync_copy(tab_hbm, tab_v)
    pltpu.sync_copy(nll_hbm, nll_v)
    pltpu.sync_copy(inp_hbm.at[pl.ds(base, TPW)], idx_v)
    pltpu.sync_copy(tgt_hbm.at[pl.ds(base, TPW)], tgt_v)

    def loss_group(g, acc):
        iv = idx_v[pl.ds(g * L, L)]
        tv = tgt_v[pl.ds(g * L, L)]
        return acc + plsc.load_gather(nll_v, [iv * V + tv])

    total = lax.fori_loop(0, TPW // L, loss_group, jnp.zeros((L,), jnp.float32))
    part_v[...] = total * (1.0 / N)
    pltpu.sync_copy(part_v, part_hbm.at[wid])

    bufs = (rows_a, rows_b)
    sems = (sem_a, sem_b)
    pending = [None, None]
    lane = lax.iota(jnp.int32, L)
    for c in range(NCH):
        slot = c % 2
        buf = bufs[slot]
        if pending[slot] is not None:
            pending[slot].wait()

        def build(g, carry):
            iv = idx_v[pl.ds(c * CH + g * L, L)]
            srcb = iv * V
            row = g * L + lane
            for v in range(V):
                vals = plsc.load_gather(tab_v, [srcb + v])
                plsc.store_scatter(buf, [row, lane * 0 + v], vals)
            return carry

        lax.fori_loop(0, CH // L, build, 0)
        bb = base // T + c // CPB
        t0 = (c % CPB) * CH
        pending[slot] = pltpu.async_copy(
            buf, out_hbm.at[bb, pl.ds(t0, CH)], sems[slot],
        )
    pending[0].wait()
    pending[1].wait()


def kernel(input, target, table):
    nll = _nll_table(table).reshape(V * V)
    logits, parts = _sc_body(
        input.reshape(N), target.reshape(N), table.reshape(V * V), nll
    )
    return logits, jnp.sum(parts)


# speedup vs baseline: 2.2443x; 2.2443x over previous
import functools

import jax
import jax.numpy as jnp
from jax import lax
from jax.experimental import pallas as pl
from jax.experimental.pallas import tpu as pltpu
from jax.experimental.pallas import tpu_sc as plsc

V = 65
B, T = 64, 2048
N = B * T

NC, NS, L = 2, 16, 16
NW = NC * NS
BG, TG = 8, 4
BB, TB = B // BG, T // TG
VC = 5
NSL = V // VC


def _prep_body(table_ref, nll_ref, tabt_ref):
    x = table_ref[...]
    m = jnp.max(x, axis=-1, keepdims=True)
    lse = m + jnp.log(jnp.sum(jnp.exp(x - m), axis=-1, keepdims=True))
    nll_ref[...] = lse - x
    tabt_ref[...] = x.T


_prep = pl.pallas_call(
    _prep_body,
    out_shape=(
        jax.ShapeDtypeStruct((V, V), jnp.float32),
        jax.ShapeDtypeStruct((V, V), jnp.float32),
    ),
)


_sc_mesh = plsc.VectorSubcoreMesh(
    core_axis_name="c", subcore_axis_name="s", num_cores=NC, num_subcores=NS
)


@functools.partial(
    pl.kernel,
    out_type=(
        jax.ShapeDtypeStruct((V, B, T), jnp.float32),
        jax.ShapeDtypeStruct((NW, L), jnp.float32),
    ),
    mesh=_sc_mesh,
    compiler_params=pltpu.CompilerParams(
        needs_layout_passes=False, use_tc_tiling_on_sc=True
    ),
    scratch_types=[
        pltpu.VMEM((BB, TB), jnp.int32),
        pltpu.VMEM((BB, TB), jnp.int32),
        pltpu.VMEM((VC, BB, TB), jnp.float32),
        pltpu.VMEM((VC, BB, TB), jnp.float32),
        pltpu.VMEM((V * V,), jnp.float32),
        pltpu.VMEM((V * V,), jnp.float32),
        pltpu.VMEM((L,), jnp.float32),
        pltpu.SemaphoreType.DMA,
        pltpu.SemaphoreType.DMA,
    ],
)
def _sc_body(inp_hbm, tgt_hbm, tabt_hbm, nll_hbm, out_hbm, part_hbm,
             idx_v, tgt_v, slab_a, slab_b, tabt_v, nll_v, part_v,
             sem_a, sem_b):
    wid = lax.axis_index("s") * NC + lax.axis_index("c")
    bg = wid // TG
    tg = wid - bg * TG
    b0 = bg * BB
    t0 = tg * TB

    pltpu.sync_copy(tabt_hbm, tabt_v)
    pltpu.sync_copy(nll_hbm, nll_v)
    pltpu.sync_copy(inp_hbm.at[pl.ds(b0, BB), pl.ds(t0, TB)], idx_v)
    pltpu.sync_copy(tgt_hbm.at[pl.ds(b0, BB), pl.ds(t0, TB)], tgt_v)

    def loss_group(g, acc):
        bq = g // (TB // L)
        tq = g - bq * (TB // L)
        iv = idx_v[bq, pl.ds(tq * L, L)]
        tv = tgt_v[bq, pl.ds(tq * L, L)]
        return acc + plsc.load_gather(nll_v, [iv * V + tv])

    total = lax.fori_loop(
        0, (BB * TB) // L, loss_group, jnp.zeros((L,), jnp.float32)
    )
    part_v[...] = total * (1.0 / N)
    pltpu.sync_copy(part_v, part_hbm.at[wid])

    bufs = (slab_a, slab_b)
    sems = (sem_a, sem_b)
    pending = [None, None]
    for s in range(NSL):
        slot = s % 2
        buf = bufs[slot]
        if pending[slot] is not None:
            pending[slot].wait()

        def build(g, carry, _s=s, _buf=buf):
            bq = g // (TB // L)
            tq = g - bq * (TB // L)
            iv = idx_v[bq, pl.ds(tq * L, L)]
            for vi in range(VC):
                vals = plsc.load_gather(tabt_v, [iv + (_s * VC + vi) * V])
                _buf[vi, bq, pl.ds(tq * L, L)] = vals
            return carry

        lax.fori_loop(0, (BB * TB) // L, build, 0)
        pending[slot] = pltpu.async_copy(
            buf,
            out_hbm.at[pl.ds(s * VC, VC), pl.ds(b0, BB), pl.ds(t0, TB)],
            sems[slot],
        )
    pending[0].wait()
    pending[1].wait()


def kernel(input, target, table):
    nll, tabt = _prep(table)
    logits_vbt, parts = _sc_body(
        input, target, tabt.reshape(V * V), nll.reshape(V * V)
    )
    return jnp.transpose(logits_vbt, (1, 2, 0)), jnp.sum(parts)


# speedup vs baseline: 2.3605x; 1.0518x over previous
import functools

import jax
import jax.numpy as jnp
from jax import lax
from jax.experimental import pallas as pl
from jax.experimental.pallas import tpu as pltpu
from jax.experimental.pallas import tpu_sc as plsc

V = 65
B, T = 64, 2048
N = B * T

NC, NS, L = 2, 16, 16
NW = NC * NS
BG, TG = 8, 4
BB, TB = B // BG, T // TG
VC = 5
NSL = V // VC


def _prep_body(table_ref, nll_ref, tabt_ref):
    x = table_ref[...]
    m = jnp.max(x, axis=-1, keepdims=True)
    lse = m + jnp.log(jnp.sum(jnp.exp(x - m), axis=-1, keepdims=True))
    nll_ref[...] = lse - x
    tabt_ref[...] = x.T


_prep = pl.pallas_call(
    _prep_body,
    out_shape=(
        jax.ShapeDtypeStruct((V, V), jnp.float32),
        jax.ShapeDtypeStruct((V, V), jnp.float32),
    ),
)


_sc_mesh = plsc.VectorSubcoreMesh(
    core_axis_name="c", subcore_axis_name="s", num_cores=NC, num_subcores=NS
)


@functools.partial(
    pl.kernel,
    out_type=(
        jax.ShapeDtypeStruct((V, B, T), jnp.float32),
        jax.ShapeDtypeStruct((NW, L), jnp.float32),
    ),
    mesh=_sc_mesh,
    compiler_params=pltpu.CompilerParams(
        needs_layout_passes=False, use_tc_tiling_on_sc=True,
        disable_bounds_checks=True,
    ),
    scratch_types=[
        pltpu.VMEM((BB, TB), jnp.int32),
        pltpu.VMEM((BB, TB), jnp.int32),
        pltpu.VMEM((VC, BB, TB), jnp.float32),
        pltpu.VMEM((VC, BB, TB), jnp.float32),
        pltpu.VMEM((V * V,), jnp.float32),
        pltpu.VMEM((V * V,), jnp.float32),
        pltpu.VMEM((L,), jnp.float32),
        pltpu.SemaphoreType.DMA,
        pltpu.SemaphoreType.DMA,
    ],
)
def _sc_body(inp_hbm, tgt_hbm, tabt_hbm, nll_hbm, out_hbm, part_hbm,
             idx_v, tgt_v, slab_a, slab_b, tabt_v, nll_v, part_v,
             sem_a, sem_b):
    wid = lax.axis_index("s") * NC + lax.axis_index("c")
    bg = wid // TG
    tg = wid - bg * TG
    b0 = bg * BB
    t0 = tg * TB

    pltpu.sync_copy(tabt_hbm, tabt_v)
    pltpu.sync_copy(nll_hbm, nll_v)
    pltpu.sync_copy(inp_hbm.at[pl.ds(b0, BB), pl.ds(t0, TB)], idx_v)
    pltpu.sync_copy(tgt_hbm.at[pl.ds(b0, BB), pl.ds(t0, TB)], tgt_v)

    total = jnp.zeros((L,), jnp.float32)
    for bq in range(BB):
        def loss_group(tq, acc, _bq=bq):
            iv = idx_v[_bq, pl.ds(tq * L, L)]
            tv = tgt_v[_bq, pl.ds(tq * L, L)]
            return acc + plsc.load_gather(nll_v, [iv * V + tv])

        total = lax.fori_loop(0, TB // L, loss_group, total)
    part_v[...] = total * (1.0 / N)
    pltpu.sync_copy(part_v, part_hbm.at[wid])

    bufs = (slab_a, slab_b)
    sems = (sem_a, sem_b)
    pending = [None, None]
    for s in range(NSL):
        slot = s % 2
        buf = bufs[slot]
        if pending[slot] is not None:
            pending[slot].wait()

        for bq in range(BB):
            def build(tq, carry, _s=s, _buf=buf, _bq=bq):
                iv = idx_v[_bq, pl.ds(tq * L, L)]
                for vi in range(VC):
                    _buf[vi, _bq, pl.ds(tq * L, L)] = plsc.load_gather(
                        tabt_v, [iv + (_s * VC + vi) * V]
                    )
                return carry

            lax.fori_loop(0, TB // L, build, 0)
        pending[slot] = pltpu.async_copy(
            buf,
            out_hbm.at[pl.ds(s * VC, VC), pl.ds(b0, BB), pl.ds(t0, TB)],
            sems[slot],
        )
    pending[0].wait()
    pending[1].wait()


def kernel(input, target, table):
    nll, tabt = _prep(table)
    logits_vbt, parts = _sc_body(
        input, target, tabt.reshape(V * V), nll.reshape(V * V)
    )
    return jnp.transpose(logits_vbt, (1, 2, 0)), jnp.sum(parts)


# speedup vs baseline: 5.4979x; 2.3291x over previous
import functools

import jax
import jax.numpy as jnp
from jax import lax
from jax.experimental import pallas as pl
from jax.experimental.pallas import tpu as pltpu
from jax.experimental.pallas import tpu_sc as plsc

V = 65
B, T = 64, 2048
N = B * T

NC, NS, L = 2, 16, 16
NW = NC * NS
BG, TG = 8, 4
BB, TB = B // BG, T // TG
VC = 5
NSL = V // VC


def _prep_body(table_ref, nll_ref, tabt_ref):
    x = table_ref[...]
    m = jnp.max(x, axis=-1, keepdims=True)
    lse = m + jnp.log(jnp.sum(jnp.exp(x - m), axis=-1, keepdims=True))
    nll_ref[...] = lse - x
    tabt_ref[...] = x.T


_prep = pl.pallas_call(
    _prep_body,
    out_shape=(
        jax.ShapeDtypeStruct((V, V), jnp.float32),
        jax.ShapeDtypeStruct((V, V), jnp.float32),
    ),
)


_sc_mesh = plsc.VectorSubcoreMesh(
    core_axis_name="c", subcore_axis_name="s", num_cores=NC, num_subcores=NS
)


@functools.partial(
    pl.kernel,
    out_type=(
        jax.ShapeDtypeStruct((V, B, T), jnp.float32),
        jax.ShapeDtypeStruct((NW, L), jnp.float32),
    ),
    mesh=_sc_mesh,
    compiler_params=pltpu.CompilerParams(
        needs_layout_passes=False, use_tc_tiling_on_sc=True,
        disable_bounds_checks=True,
    ),
    scratch_types=[
        pltpu.VMEM((BB, TB), jnp.int32),
        pltpu.VMEM((BB, TB), jnp.int32),
        pltpu.VMEM((VC, BB, TB), jnp.float32),
        pltpu.VMEM((VC, BB, TB), jnp.float32),
        pltpu.VMEM((V * V,), jnp.float32),
        pltpu.VMEM((V * V,), jnp.float32),
        pltpu.VMEM((L,), jnp.float32),
        pltpu.SemaphoreType.DMA,
        pltpu.SemaphoreType.DMA,
    ],
)
def _sc_body(inp_hbm, tgt_hbm, tabt_hbm, nll_hbm, out_hbm, part_hbm,
             idx_v, tgt_v, slab_a, slab_b, tabt_v, nll_v, part_v,
             sem_a, sem_b):
    wid = lax.axis_index("s") * NC + lax.axis_index("c")
    bg = wid // TG
    tg = wid - bg * TG
    b0 = bg * BB
    t0 = tg * TB

    pltpu.sync_copy(tabt_hbm, tabt_v)
    pltpu.sync_copy(nll_hbm, nll_v)
    pltpu.sync_copy(inp_hbm.at[pl.ds(b0, BB), pl.ds(t0, TB)], idx_v)
    pltpu.sync_copy(tgt_hbm.at[pl.ds(b0, BB), pl.ds(t0, TB)], tgt_v)

    total = jnp.zeros((L,), jnp.float32)
    for bq in range(BB):
        def loss_group(tq, acc, _bq=bq):
            iv = idx_v[_bq, pl.ds(tq * L, L)]
            tv = tgt_v[_bq, pl.ds(tq * L, L)]
            return acc + plsc.load_gather(nll_v, [iv * V + tv])

        total = lax.fori_loop(0, TB // L, loss_group, total)
    part_v[...] = total * (1.0 / N)
    pltpu.sync_copy(part_v, part_hbm.at[wid])

    bufs = (slab_a, slab_b)
    sems = (sem_a, sem_b)
    pending = [None, None]
    for s in range(NSL):
        slot = s % 2
        buf = bufs[slot]
        if pending[slot] is not None:
            pending[slot].wait()

        def row(bq, carry, _s=s, _buf=buf):
            def build(tq):
                iv = idx_v[bq, pl.ds(tq * L, L)]
                for vi in range(VC):
                    _buf[vi, bq, pl.ds(tq * L, L)] = plsc.load_gather(
                        tabt_v, [iv + (_s * VC + vi) * V]
                    )

            plsc.parallel_loop(0, TB // L, unroll=4)(build)
            return carry

        lax.fori_loop(0, BB, row, 0)
        pending[slot] = pltpu.async_copy(
            buf,
            out_hbm.at[pl.ds(s * VC, VC), pl.ds(b0, BB), pl.ds(t0, TB)],
            sems[slot],
        )
    pending[0].wait()
    pending[1].wait()


def kernel(input, target, table):
    nll, tabt = _prep(table)
    logits_vbt, parts = _sc_body(
        input, target, tabt.reshape(V * V), nll.reshape(V * V)
    )
    return jnp.transpose(logits_vbt, (1, 2, 0)), jnp.sum(parts)


# speedup vs baseline: 5.7207x; 1.0405x over previous
import functools

import jax
import jax.numpy as jnp
from jax import lax
from jax.experimental import pallas as pl
from jax.experimental.pallas import tpu as pltpu
from jax.experimental.pallas import tpu_sc as plsc

V = 65
B, T = 64, 2048
N = B * T

NC, NS, L = 2, 16, 16
NW = NC * NS
BG, TG = 8, 4
BB, TB = B // BG, T // TG
VC = 13
NSL = V // VC


def _prep_body(table_ref, nll_ref, tabt_ref):
    x = table_ref[...]
    m = jnp.max(x, axis=-1, keepdims=True)
    lse = m + jnp.log(jnp.sum(jnp.exp(x - m), axis=-1, keepdims=True))
    nll_ref[...] = lse - x
    tabt_ref[...] = x.T


_prep = pl.pallas_call(
    _prep_body,
    out_shape=(
        jax.ShapeDtypeStruct((V, V), jnp.float32),
        jax.ShapeDtypeStruct((V, V), jnp.float32),
    ),
)


_sc_mesh = plsc.VectorSubcoreMesh(
    core_axis_name="c", subcore_axis_name="s", num_cores=NC, num_subcores=NS
)


@functools.partial(
    pl.kernel,
    out_type=(
        jax.ShapeDtypeStruct((V, B, T), jnp.float32),
        jax.ShapeDtypeStruct((NW, L), jnp.float32),
    ),
    mesh=_sc_mesh,
    compiler_params=pltpu.CompilerParams(
        needs_layout_passes=False, use_tc_tiling_on_sc=True,
        disable_bounds_checks=True,
    ),
    scratch_types=[
        pltpu.VMEM((BB, TB), jnp.int32),
        pltpu.VMEM((BB, TB), jnp.int32),
        pltpu.VMEM((VC, BB, TB), jnp.float32),
        pltpu.VMEM((VC, BB, TB), jnp.float32),
        pltpu.VMEM((V * V,), jnp.float32),
        pltpu.VMEM((V * V,), jnp.float32),
        pltpu.VMEM((L,), jnp.float32),
        pltpu.SemaphoreType.DMA,
        pltpu.SemaphoreType.DMA,
        pltpu.SemaphoreType.DMA,
        pltpu.SemaphoreType.DMA,
    ],
)
def _sc_body(inp_hbm, tgt_hbm, tabt_hbm, nll_hbm, out_hbm, part_hbm,
             idx_v, tgt_v, slab_a, slab_b, tabt_v, nll_v, part_v,
             sem_a, sem_b, sem_c, sem_d):
    wid = lax.axis_index("s") * NC + lax.axis_index("c")
    bg = wid // TG
    tg = wid - bg * TG
    b0 = bg * BB
    t0 = tg * TB

    tab_dma = pltpu.async_copy(tabt_hbm, tabt_v, sem_a)
    idx_dma = pltpu.async_copy(
        inp_hbm.at[pl.ds(b0, BB), pl.ds(t0, TB)], idx_v, sem_b
    )
    tgt_dma = pltpu.async_copy(
        tgt_hbm.at[pl.ds(b0, BB), pl.ds(t0, TB)], tgt_v, sem_c
    )
    nll_dma = pltpu.async_copy(nll_hbm, nll_v, sem_d)
    tab_dma.wait()
    idx_dma.wait()

    bufs = (slab_a, slab_b)
    sems = (sem_a, sem_b)
    pending = [None, None]
    for s in range(NSL):
        slot = s % 2
        buf = bufs[slot]
        if pending[slot] is not None:
            pending[slot].wait()

        def row(bq, carry, _s=s, _buf=buf):
            def build(tq):
                iv = idx_v[bq, pl.ds(tq * L, L)]
                for vi in range(VC):
                    _buf[vi, bq, pl.ds(tq * L, L)] = plsc.load_gather(
                        tabt_v, [iv + (_s * VC + vi) * V]
                    )

            plsc.parallel_loop(0, TB // L, unroll=8)(build)
            return carry

        lax.fori_loop(0, BB, row, 0)
        pending[slot] = pltpu.async_copy(
            buf,
            out_hbm.at[pl.ds(s * VC, VC), pl.ds(b0, BB), pl.ds(t0, TB)],
            sems[slot],
        )

    tgt_dma.wait()
    nll_dma.wait()
    total = jnp.zeros((L,), jnp.float32)
    for bq in range(BB):
        def loss_group(tq, acc, _bq=bq):
            iv = idx_v[_bq, pl.ds(tq * L, L)]
            tv = tgt_v[_bq, pl.ds(tq * L, L)]
            return acc + plsc.load_gather(nll_v, [iv * V + tv])

        total = lax.fori_loop(0, TB // L, loss_group, total)
    part_v[...] = total * (1.0 / N)
    pltpu.sync_copy(part_v, part_hbm.at[wid])

    pending[0].wait()
    pending[1].wait()


def kernel(input, target, table):
    nll, tabt = _prep(table)
    logits_vbt, parts = _sc_body(
        input, target, tabt.reshape(V * V), nll.reshape(V * V)
    )
    return jnp.transpose(logits_vbt, (1, 2, 0)), jnp.sum(parts)
